# baseline (speedup 1.0000x reference)
import jax
import jax.numpy as jnp
from jax.experimental import pallas as pl
from jax.experimental.pallas import tpu as pltpu

_NT = 4


def _body(bb, L, C, O):
    F = 3 * O

    CH = min(8, bb)

    def body(x_ref, w_ref, b_ref, fcw_ref, fcb_ref, out_ref):
        t_idx = (L - 8) + jax.lax.broadcasted_iota(jnp.int32, (8, F), 0)
        blk = jax.lax.broadcasted_iota(jnp.int32, (8, F), 1) // O
        valid = t_idx < (L - 1 - blk)

        feats = []
        for c0 in range(0, bb, CH):
            cols = []
            for b in range(c0, c0 + CH):
                xb = x_ref[b].astype(jnp.bfloat16)
                parts = [xb]
                for j in range(1, _NT):
                    parts.append(
                        jnp.concatenate([xb[:, j:], xb[:, :j]], axis=1))
                cols.append(jnp.concatenate(parts, axis=0))
            xcat = jnp.concatenate(cols, axis=1)
            y = jax.lax.dot_general(
                xcat, w_ref[...],
                dimension_numbers=(((0,), (0,)), ((), ())),
                preferred_element_type=jnp.float32)

            for b in range(CH):
                yb = y[b * L:(b + 1) * L]
                body_max = jnp.max(yb[:L - 8], axis=0, keepdims=True)
                tail = jnp.where(valid, yb[L - 8:], -jnp.inf)
                tail_max = jnp.max(tail, axis=0, keepdims=True)
                feats.append(jnp.maximum(body_max, tail_max))

        feat = jnp.maximum(jnp.concatenate(feats, axis=0) + b_ref[...], 0.0)
        logits = jnp.dot(feat, fcw_ref[...],
                         preferred_element_type=jnp.float32) + fcb_ref[...]
        m = jnp.max(logits, axis=1, keepdims=True)
        e = jnp.exp(logits - m)
        s = jnp.sum(e, axis=1, keepdims=True)
        out_ref[...] = e * pl.reciprocal(s, approx=True)

    return body


def kernel(x_ncl, w2, b2, w3, b3, w4, b4, fc_w, fc_b):
    B, C, L = x_ncl.shape
    O = w2.shape[0]
    F = 3 * O

    zeros = jnp.zeros((C, O), jnp.float32)

    def tap(j):
        cols = [jnp.transpose(w[:, :, j], (1, 0)) if j < k else zeros
                for w, k in ((w2, 2), (w3, 3), (w4, 4))]
        return jnp.concatenate(cols, axis=1)

    w_cat = jnp.concatenate([tap(j) for j in range(_NT)],
                            axis=0).astype(jnp.bfloat16)
    bias = jnp.concatenate([b2, b3, b4]).reshape(1, F)
    fcw = jnp.transpose(fc_w, (1, 0))
    fcb = fc_b.reshape(1, 2)

    bb = next((c for c in (32, 16, 8) if B % c == 0), B)
    grid = (B // bb,)

    return pl.pallas_call(
        _body(bb, L, C, O),
        out_shape=jax.ShapeDtypeStruct((B, 2), jnp.float32),
        grid=grid,
        in_specs=[
            pl.BlockSpec((bb, C, L), lambda i: (i, 0, 0)),
            pl.BlockSpec((_NT * C, F), lambda i: (0, 0)),
            pl.BlockSpec((1, F), lambda i: (0, 0)),
            pl.BlockSpec((F, 2), lambda i: (0, 0)),
            pl.BlockSpec((1, 2), lambda i: (0, 0)),
        ],
        out_specs=pl.BlockSpec((bb, 2), lambda i: (i, 0)),
        compiler_params=pltpu.CompilerParams(
            dimension_semantics=("parallel",)),
    )(x_ncl, w_cat, bias, fcw, fcb)


# speedup vs baseline: 1.0137x; 1.0137x over previous
import jax
import jax.numpy as jnp
from jax.experimental import pallas as pl
from jax.experimental.pallas import tpu as pltpu

_NT = 4


def _body(bb, L, C, O):
    F = 3 * O

    CH = min(2, bb)

    def body(x_ref, w_ref, b_ref, fcw_ref, fcb_ref, out_ref):
        t_idx = (L - 8) + jax.lax.broadcasted_iota(jnp.int32, (8, F), 0)
        blk = jax.lax.broadcasted_iota(jnp.int32, (8, F), 1) // O
        valid = t_idx < (L - 1 - blk)

        feats = []
        for c0 in range(0, bb, CH):
            cols = []
            for b in range(c0, c0 + CH):
                xb = x_ref[b].astype(jnp.bfloat16)
                parts = [xb]
                for j in range(1, _NT):
                    parts.append(
                        jnp.concatenate([xb[:, j:], xb[:, :j]], axis=1))
                cols.append(jnp.concatenate(parts, axis=0))
            xcat = jnp.concatenate(cols, axis=1)
            y = jax.lax.dot_general(
                xcat, w_ref[...],
                dimension_numbers=(((0,), (0,)), ((), ())),
                preferred_element_type=jnp.float32)

            for b in range(CH):
                yb = y[b * L:(b + 1) * L]
                body_max = jnp.max(yb[:L - 8], axis=0, keepdims=True)
                tail = jnp.where(valid, yb[L - 8:], -jnp.inf)
                tail_max = jnp.max(tail, axis=0, keepdims=True)
                feats.append(jnp.maximum(body_max, tail_max))

        feat = jnp.maximum(jnp.concatenate(feats, axis=0) + b_ref[...], 0.0)
        logits = jnp.dot(feat, fcw_ref[...],
                         preferred_element_type=jnp.float32) + fcb_ref[...]
        m = jnp.max(logits, axis=1, keepdims=True)
        e = jnp.exp(logits - m)
        s = jnp.sum(e, axis=1, keepdims=True)
        out_ref[...] = e * pl.reciprocal(s, approx=True)

    return body


def kernel(x_ncl, w2, b2, w3, b3, w4, b4, fc_w, fc_b):
    B, C, L = x_ncl.shape
    O = w2.shape[0]
    F = 3 * O

    zeros = jnp.zeros((C, O), jnp.float32)

    def tap(j):
        cols = [jnp.transpose(w[:, :, j], (1, 0)) if j < k else zeros
                for w, k in ((w2, 2), (w3, 3), (w4, 4))]
        return jnp.concatenate(cols, axis=1)

    w_cat = jnp.concatenate([tap(j) for j in range(_NT)],
                            axis=0).astype(jnp.bfloat16)
    bias = jnp.concatenate([b2, b3, b4]).reshape(1, F)
    fcw = jnp.transpose(fc_w, (1, 0))
    fcb = fc_b.reshape(1, 2)

    bb = next((c for c in (32, 16, 8) if B % c == 0), B)
    grid = (B // bb,)

    return pl.pallas_call(
        _body(bb, L, C, O),
        out_shape=jax.ShapeDtypeStruct((B, 2), jnp.float32),
        grid=grid,
        in_specs=[
            pl.BlockSpec((bb, C, L), lambda i: (i, 0, 0)),
            pl.BlockSpec((_NT * C, F), lambda i: (0, 0)),
            pl.BlockSpec((1, F), lambda i: (0, 0)),
            pl.BlockSpec((F, 2), lambda i: (0, 0)),
            pl.BlockSpec((1, 2), lambda i: (0, 0)),
        ],
        out_specs=pl.BlockSpec((bb, 2), lambda i: (i, 0)),
        compiler_params=pltpu.CompilerParams(
            dimension_semantics=("parallel",)),
    )(x_ncl, w_cat, bias, fcw, fcb)


# speedup vs baseline: 1.0269x; 1.0131x over previous
import jax
import jax.numpy as jnp
from jax.experimental import pallas as pl
from jax.experimental.pallas import tpu as pltpu

_NT = 4


def _body(bb, L, C, O):
    F = 3 * O

    CH = min(2, bb)

    def body(x_ref, w_ref, b_ref, fcw_ref, fcb_ref, out_ref):
        t_idx = (L - 8) + jax.lax.broadcasted_iota(jnp.int32, (8, F), 0)
        blk = jax.lax.broadcasted_iota(jnp.int32, (8, F), 1) // O
        valid = t_idx < (L - 1 - blk)

        feats = []
        for c0 in range(0, bb, CH):
            cols = []
            for b in range(c0, c0 + CH):
                xb = x_ref[b].astype(jnp.bfloat16)
                parts = [xb]
                for j in range(1, _NT):
                    parts.append(
                        jnp.concatenate([xb[:, j:], xb[:, :j]], axis=1))
                cols.append(jnp.concatenate(parts, axis=0))
            xcat = jnp.concatenate(cols, axis=1)
            y = jax.lax.dot_general(
                xcat, w_ref[...],
                dimension_numbers=(((0,), (0,)), ((), ())),
                preferred_element_type=jnp.float32)

            for b in range(CH):
                yb = y[b * L:(b + 1) * L]
                body_max = jnp.max(yb[:L - 8], axis=0, keepdims=True)
                tail = jnp.where(valid, yb[L - 8:], -jnp.inf)
                tail_max = jnp.max(tail, axis=0, keepdims=True)
                feats.append(jnp.maximum(body_max, tail_max))

        feat = jnp.maximum(jnp.concatenate(feats, axis=0) + b_ref[...], 0.0)
        logits = jnp.dot(feat, fcw_ref[...],
                         preferred_element_type=jnp.float32) + fcb_ref[...]
        m = jnp.max(logits, axis=1, keepdims=True)
        e = jnp.exp(logits - m)
        s = jnp.sum(e, axis=1, keepdims=True)
        out_ref[...] = e * pl.reciprocal(s, approx=True)

    return body


def kernel(x_ncl, w2, b2, w3, b3, w4, b4, fc_w, fc_b):
    B, C, L = x_ncl.shape
    O = w2.shape[0]
    F = 3 * O

    wall = jnp.concatenate([w2, w3, w4], axis=2)
    wt = jnp.transpose(wall, (2, 1, 0))
    zeros = jnp.zeros((C, O), jnp.float32)

    def tap(j):
        cols = [wt[j] if j < 2 else zeros,
                wt[2 + j] if j < 3 else zeros,
                wt[5 + j]]
        return jnp.concatenate(cols, axis=1)

    w_cat = jnp.concatenate([tap(j) for j in range(_NT)],
                            axis=0).astype(jnp.bfloat16)
    bias = jnp.concatenate([b2, b3, b4]).reshape(1, F)
    fcw = jnp.transpose(fc_w, (1, 0))
    fcb = fc_b.reshape(1, 2)

    bb = next((c for c in (64, 32, 16, 8) if B % c == 0), B)
    grid = (B // bb,)

    return pl.pallas_call(
        _body(bb, L, C, O),
        out_shape=jax.ShapeDtypeStruct((B, 2), jnp.float32),
        grid=grid,
        in_specs=[
            pl.BlockSpec((bb, C, L), lambda i: (i, 0, 0)),
            pl.BlockSpec((_NT * C, F), lambda i: (0, 0)),
            pl.BlockSpec((1, F), lambda i: (0, 0)),
            pl.BlockSpec((F, 2), lambda i: (0, 0)),
            pl.BlockSpec((1, 2), lambda i: (0, 0)),
        ],
        out_specs=pl.BlockSpec((bb, 2), lambda i: (i, 0)),
        compiler_params=pltpu.CompilerParams(
            dimension_semantics=("parallel",)),
    )(x_ncl, w_cat, bias, fcw, fcb)


# speedup vs baseline: 1.0420x; 1.0147x over previous
import jax
import jax.numpy as jnp
from jax.experimental import pallas as pl
from jax.experimental.pallas import tpu as pltpu

_NT = 4


def _body(bb, L, C, O):
    F = 3 * O

    CH = min(2, bb)

    def body(x_ref, wt_ref, b_ref, fcw_ref, fcb_ref, out_ref):
        zeros = jnp.zeros((C, O), jnp.bfloat16)

        def tap(j):
            cols = [wt_ref[j].astype(jnp.bfloat16) if j < 2 else zeros,
                    wt_ref[2 + j].astype(jnp.bfloat16) if j < 3 else zeros,
                    wt_ref[5 + j].astype(jnp.bfloat16)]
            return jnp.concatenate(cols, axis=1)

        w = jnp.concatenate([tap(j) for j in range(_NT)], axis=0)

        t_idx = (L - 8) + jax.lax.broadcasted_iota(jnp.int32, (8, F), 0)
        blk = jax.lax.broadcasted_iota(jnp.int32, (8, F), 1) // O
        valid = t_idx < (L - 1 - blk)

        feats = []
        for c0 in range(0, bb, CH):
            cols = []
            for b in range(c0, c0 + CH):
                xb = x_ref[b].astype(jnp.bfloat16)
                parts = [xb]
                for j in range(1, _NT):
                    parts.append(
                        jnp.concatenate([xb[:, j:], xb[:, :j]], axis=1))
                cols.append(jnp.concatenate(parts, axis=0))
            xcat = jnp.concatenate(cols, axis=1)
            y = jax.lax.dot_general(
                xcat, w,
                dimension_numbers=(((0,), (0,)), ((), ())),
                preferred_element_type=jnp.float32)

            for b in range(CH):
                yb = y[b * L:(b + 1) * L]
                body_max = jnp.max(yb[:L - 8], axis=0, keepdims=True)
                tail = jnp.where(valid, yb[L - 8:], -jnp.inf)
                tail_max = jnp.max(tail, axis=0, keepdims=True)
                feats.append(jnp.maximum(body_max, tail_max))

        feat = jnp.maximum(jnp.concatenate(feats, axis=0) + b_ref[...], 0.0)
        logits = jax.lax.dot_general(
            feat, fcw_ref[...],
            dimension_numbers=(((1,), (1,)), ((), ())),
            preferred_element_type=jnp.float32) + fcb_ref[...]
        m = jnp.max(logits, axis=1, keepdims=True)
        e = jnp.exp(logits - m)
        s = jnp.sum(e, axis=1, keepdims=True)
        out_ref[...] = e * pl.reciprocal(s, approx=True)

    return body


def kernel(x_ncl, w2, b2, w3, b3, w4, b4, fc_w, fc_b):
    B, C, L = x_ncl.shape
    O = w2.shape[0]
    F = 3 * O

    wall = jnp.concatenate([w2, w3, w4], axis=2)
    wt = jnp.transpose(wall, (2, 1, 0))
    bias = jnp.concatenate([b2, b3, b4]).reshape(1, F)
    fcb = fc_b.reshape(1, 2)

    bb = next((c for c in (64, 32, 16, 8) if B % c == 0), B)
    grid = (B // bb,)

    return pl.pallas_call(
        _body(bb, L, C, O),
        out_shape=jax.ShapeDtypeStruct((B, 2), jnp.float32),
        grid=grid,
        in_specs=[
            pl.BlockSpec((bb, C, L), lambda i: (i, 0, 0)),
            pl.BlockSpec((3 * _NT - 3, C, O), lambda i: (0, 0, 0)),
            pl.BlockSpec((1, F), lambda i: (0, 0)),
            pl.BlockSpec((2, F), lambda i: (0, 0)),
            pl.BlockSpec((1, 2), lambda i: (0, 0)),
        ],
        out_specs=pl.BlockSpec((bb, 2), lambda i: (i, 0)),
        compiler_params=pltpu.CompilerParams(
            dimension_semantics=("parallel",)),
    )(x_ncl, wt, bias, fc_w, fcb)
